# baseline (speedup 1.0000x reference)
import functools

import jax
import jax.numpy as jnp
from jax import lax
from jax.experimental import pallas as pl
from jax.experimental.pallas import tpu as pltpu
from jax.experimental.pallas import tpu_sc as plsc

B = 16384
V = 1000000
R = 64
RP = 128
ALPHA = 0.01

NC = 2
NS = 16
NW = NC * NS
BPW = B // NW
HB = BPW // 2

XBK = 2048


def _xpose_body(in_ref, eye_ref, out_ref):
    out_ref[...] = lax.dot_general(
        in_ref[...], eye_ref[...], (((0,), (0,)), ((), ())),
        preferred_element_type=jnp.float32)


@functools.lru_cache(maxsize=None)
def _make_xpose():
    grid = (V + XBK - 1) // XBK
    return pl.pallas_call(
        _xpose_body,
        grid=(grid,),
        in_specs=[pl.BlockSpec((R, XBK), lambda i: (0, i)),
                  pl.BlockSpec((R, R), lambda i: (0, 0))],
        out_specs=pl.BlockSpec((XBK, R), lambda i: (i, 0)),
        out_shape=jax.ShapeDtypeStruct((V, R), jnp.float32),
        compiler_params=pltpu.CompilerParams(
            fuse_transposed_lhs_in_matmul=True),
    )


def _sc_gather_body(k_hbm, emb0_hbm, emb1_hbm, e0_hbm, e1_hbm,
                    idx_vmem, buf0, buf1, sem0, sem1):
    wid = lax.axis_index("s") * NC + lax.axis_index("c")
    base = wid * BPW
    pltpu.sync_copy(k_hbm.at[pl.ds(base, BPW)], idx_vmem.at[pl.ds(0, BPW)])
    pltpu.sync_copy(k_hbm.at[pl.ds(B + base, BPW)],
                    idx_vmem.at[pl.ds(BPW, BPW)])

    def fire(voff, emb, buf, sem):
        def gath(g, carry):
            v = idx_vmem[pl.ds(voff + g * 16, 16)]
            for j in range(16):
                pltpu.async_copy(emb.at[pl.ds(v[j], 1)],
                                 buf.at[pl.ds(g * 16 + j, 1)], sem)
            return carry
        lax.fori_loop(0, HB // 16, gath, 0)

    def drain(emb, buf, sem):
        pltpu.make_async_copy(emb.at[pl.ds(0, HB)], buf, sem).wait()

    units = [(0, emb0_hbm, e0_hbm, 0), (HB, emb0_hbm, e0_hbm, HB),
             (BPW, emb1_hbm, e1_hbm, 0), (BPW + HB, emb1_hbm, e1_hbm, HB)]
    bufs = (buf0, buf1)
    sems = (sem0, sem1)
    for u, (voff, emb, _, _) in enumerate(units):
        s = u % 2
        if u >= 2:
            pemb, pout, poff = units[u - 2][1], units[u - 2][2], units[u - 2][3]
            drain(pemb, bufs[s], sems[s])
            pltpu.sync_copy(bufs[s], pout.at[pl.ds(base + poff, HB)])
        fire(voff, emb, bufs[s], sems[s])
    for u in (2, 3):
        s = u % 2
        emb, out, off = units[u][1], units[u][2], units[u][3]
        drain(emb, bufs[s], sems[s])
        pltpu.sync_copy(bufs[s], out.at[pl.ds(base + off, HB)])


@functools.lru_cache(maxsize=None)
def _make_sc_gather():
    return pl.kernel(
        _sc_gather_body,
        out_type=(jax.ShapeDtypeStruct((B, R), jnp.float32),
                  jax.ShapeDtypeStruct((B, R), jnp.float32)),
        mesh=plsc.VectorSubcoreMesh(core_axis_name="c", subcore_axis_name="s",
                                    num_cores=NC, num_subcores=NS),
        scratch_types=[
            pltpu.VMEM((2 * BPW,), jnp.int32),
            pltpu.VMEM((HB, R), jnp.float32),
            pltpu.VMEM((HB, R), jnp.float32),
            pltpu.SemaphoreType.DMA,
            pltpu.SemaphoreType.DMA,
        ],
        compiler_params=pltpu.CompilerParams(use_tc_tiling_on_sc=True),
    )


def _dense_body(ll_ref, e0_ref, e1_ref, w0_ref, a0_ref, a1_ref, b1_ref,
                wmu_ref, bmu_ref, wlv_ref, blv_ref, mu_ref, lv_ref):
    e0 = e0_ref[...]
    e1 = e1_ref[...]
    h = (ll_ref[...] * w0_ref[...]
         + jnp.dot(e0, a0_ref[...], preferred_element_type=jnp.float32)
         + jnp.dot(e1, a1_ref[...], preferred_element_type=jnp.float32)
         + b1_ref[...])
    h = jnp.where(h >= 0, h, ALPHA * h)
    mu_ref[...] = (jnp.dot(h, wmu_ref[...], preferred_element_type=jnp.float32)
                   + bmu_ref[...] + e0 + e1)
    lv_ref[...] = (jnp.dot(h, wlv_ref[...], preferred_element_type=jnp.float32)
                   + blv_ref[...])


def _dense(ll, e0, e1, w0, a0, a1, b1, wmu, bmu, wlv, blv, blk=2048):
    grid = B // blk
    row_spec = lambda w: pl.BlockSpec((blk, w), lambda i: (i, 0))
    full = lambda s: pl.BlockSpec(s, lambda i: (0, 0))
    return pl.pallas_call(
        _dense_body,
        grid=(grid,),
        in_specs=[
            row_spec(1), row_spec(R), row_spec(R),
            full((1, RP)), full((R, RP)), full((R, RP)), full((1, RP)),
            full((RP, R)), full((1, R)), full((RP, R)), full((1, R)),
        ],
        out_specs=[row_spec(R), row_spec(R)],
        out_shape=[jax.ShapeDtypeStruct((B, R), jnp.float32),
                   jax.ShapeDtypeStruct((B, R), jnp.float32)],
    )(ll, e0, e1, w0, a0, a1, b1, wmu, bmu, wlv, blv)


def kernel(log_lib, K, emb0, emb1, W1, b1, Wmu, bmu, Wlv, blv):
    xpose = _make_xpose()
    eye = jnp.eye(R, dtype=jnp.float32)
    emb0r = xpose(emb0.T, eye)
    emb1r = xpose(emb1.T, eye)
    e0, e1 = _make_sc_gather()(K.reshape(2 * B), emb0r, emb1r)
    w0 = W1[:, 0:1].T
    a0 = W1[:, 1:1 + R].T
    a1 = W1[:, 1 + R:1 + 2 * R].T
    mu, lv = _dense(log_lib.reshape(B, 1), e0, e1, w0, a0, a1,
                    b1.reshape(1, RP), Wmu.T, bmu.reshape(1, R),
                    Wlv.T, blv.reshape(1, R))
    return mu, lv


# speedup vs baseline: 1.7134x; 1.7134x over previous
import functools

import jax
import jax.numpy as jnp
from jax import lax
from jax.experimental import pallas as pl
from jax.experimental.pallas import tpu as pltpu
from jax.experimental.pallas import tpu_sc as plsc

B = 16384
V = 1000000
R = 64
RP = 128
ALPHA = 0.01

NC = 2
NS = 16
NW = NC * NS
BPW = B // NW
HB = BPW // 2

XBK = 8192


def _xpose_body(in_ref, eye_ref, out_ref):
    out_ref[...] = lax.dot_general(
        in_ref[...], eye_ref[...], (((0,), (0,)), ((), ())),
        preferred_element_type=jnp.float32)


@functools.lru_cache(maxsize=None)
def _make_xpose():
    grid = (V + XBK - 1) // XBK
    return pl.pallas_call(
        _xpose_body,
        grid=(grid,),
        in_specs=[pl.BlockSpec((R, XBK), lambda i: (0, i)),
                  pl.BlockSpec((R, R), lambda i: (0, 0))],
        out_specs=pl.BlockSpec((XBK, R), lambda i: (i, 0)),
        out_shape=jax.ShapeDtypeStruct((V, R), jnp.float32),
        compiler_params=pltpu.CompilerParams(
            fuse_transposed_lhs_in_matmul=True),
    )


def _sc_gather_body(k_hbm, emb0_hbm, emb1_hbm, e0_hbm, e1_hbm,
                    idx_vmem, buf0, buf1, sem0, sem1):
    wid = lax.axis_index("s") * NC + lax.axis_index("c")
    base = wid * BPW
    pltpu.sync_copy(k_hbm.at[pl.ds(base, BPW)], idx_vmem.at[pl.ds(0, BPW)])
    pltpu.sync_copy(k_hbm.at[pl.ds(B + base, BPW)],
                    idx_vmem.at[pl.ds(BPW, BPW)])

    def fire(voff, emb, buf, sem):
        def gath(g, carry):
            v = idx_vmem[pl.ds(voff + g * 16, 16)]
            for j in range(16):
                pltpu.async_copy(emb.at[pl.ds(v[j], 1)],
                                 buf.at[pl.ds(g * 16 + j, 1)], sem)
            return carry
        lax.fori_loop(0, HB // 16, gath, 0)

    def drain(emb, buf, sem):
        pltpu.make_async_copy(emb.at[pl.ds(0, HB)], buf, sem).wait()

    units = [(0, emb0_hbm, e0_hbm, 0), (HB, emb0_hbm, e0_hbm, HB),
             (BPW, emb1_hbm, e1_hbm, 0), (BPW + HB, emb1_hbm, e1_hbm, HB)]
    bufs = (buf0, buf1)
    sems = (sem0, sem1)
    for u, (voff, emb, _, _) in enumerate(units):
        s = u % 2
        if u >= 2:
            pemb, pout, poff = units[u - 2][1], units[u - 2][2], units[u - 2][3]
            drain(pemb, bufs[s], sems[s])
            pltpu.sync_copy(bufs[s], pout.at[pl.ds(base + poff, HB)])
        fire(voff, emb, bufs[s], sems[s])
    for u in (2, 3):
        s = u % 2
        emb, out, off = units[u][1], units[u][2], units[u][3]
        drain(emb, bufs[s], sems[s])
        pltpu.sync_copy(bufs[s], out.at[pl.ds(base + off, HB)])


@functools.lru_cache(maxsize=None)
def _make_sc_gather():
    return pl.kernel(
        _sc_gather_body,
        out_type=(jax.ShapeDtypeStruct((B, R), jnp.float32),
                  jax.ShapeDtypeStruct((B, R), jnp.float32)),
        mesh=plsc.VectorSubcoreMesh(core_axis_name="c", subcore_axis_name="s",
                                    num_cores=NC, num_subcores=NS),
        scratch_types=[
            pltpu.VMEM((2 * BPW,), jnp.int32),
            pltpu.VMEM((HB, R), jnp.float32),
            pltpu.VMEM((HB, R), jnp.float32),
            pltpu.SemaphoreType.DMA,
            pltpu.SemaphoreType.DMA,
        ],
        compiler_params=pltpu.CompilerParams(use_tc_tiling_on_sc=True),
    )


def _dense_body(ll_ref, e0_ref, e1_ref, w0_ref, a0_ref, a1_ref, b1_ref,
                wmu_ref, bmu_ref, wlv_ref, blv_ref, mu_ref, lv_ref):
    e0 = e0_ref[...]
    e1 = e1_ref[...]
    h = (ll_ref[...] * w0_ref[...]
         + jnp.dot(e0, a0_ref[...], preferred_element_type=jnp.float32)
         + jnp.dot(e1, a1_ref[...], preferred_element_type=jnp.float32)
         + b1_ref[...])
    h = jnp.where(h >= 0, h, ALPHA * h)
    mu_ref[...] = (jnp.dot(h, wmu_ref[...], preferred_element_type=jnp.float32)
                   + bmu_ref[...] + e0 + e1)
    lv_ref[...] = (jnp.dot(h, wlv_ref[...], preferred_element_type=jnp.float32)
                   + blv_ref[...])


def _dense(ll, e0, e1, w0, a0, a1, b1, wmu, bmu, wlv, blv, blk=2048):
    grid = B // blk
    row_spec = lambda w: pl.BlockSpec((blk, w), lambda i: (i, 0))
    full = lambda s: pl.BlockSpec(s, lambda i: (0, 0))
    return pl.pallas_call(
        _dense_body,
        grid=(grid,),
        in_specs=[
            row_spec(1), row_spec(R), row_spec(R),
            full((1, RP)), full((R, RP)), full((R, RP)), full((1, RP)),
            full((RP, R)), full((1, R)), full((RP, R)), full((1, R)),
        ],
        out_specs=[row_spec(R), row_spec(R)],
        out_shape=[jax.ShapeDtypeStruct((B, R), jnp.float32),
                   jax.ShapeDtypeStruct((B, R), jnp.float32)],
    )(ll, e0, e1, w0, a0, a1, b1, wmu, bmu, wlv, blv)


def kernel(log_lib, K, emb0, emb1, W1, b1, Wmu, bmu, Wlv, blv):
    xpose = _make_xpose()
    eye = jnp.eye(R, dtype=jnp.float32)
    emb0r = xpose(emb0.T, eye)
    emb1r = xpose(emb1.T, eye)
    e0, e1 = _make_sc_gather()(K.reshape(2 * B), emb0r, emb1r)
    w0 = W1[:, 0:1].T
    a0 = W1[:, 1:1 + R].T
    a1 = W1[:, 1 + R:1 + 2 * R].T
    mu, lv = _dense(log_lib.reshape(B, 1), e0, e1, w0, a0, a1,
                    b1.reshape(1, RP), Wmu.T, bmu.reshape(1, R),
                    Wlv.T, blv.reshape(1, R))
    return mu, lv


# speedup vs baseline: 1.9119x; 1.1159x over previous
import functools

import jax
import jax.numpy as jnp
from jax import lax
from jax.experimental import pallas as pl
from jax.experimental.pallas import tpu as pltpu
from jax.experimental.pallas import tpu_sc as plsc

B = 16384
V = 1000000
R = 64
RP = 128
ALPHA = 0.01

NC = 2
NS = 16
NW = NC * NS
BPW = B // NW
HB = BPW // 2

XBK = 32000


def _xpose_body(in_ref, eye_ref, out_ref):
    out_ref[...] = lax.dot_general(
        in_ref[...], eye_ref[...], (((0,), (0,)), ((), ())),
        preferred_element_type=jnp.float32)


@functools.lru_cache(maxsize=None)
def _make_xpose():
    grid = (V + XBK - 1) // XBK
    return pl.pallas_call(
        _xpose_body,
        grid=(grid,),
        in_specs=[pl.BlockSpec((R, XBK), lambda i: (0, i)),
                  pl.BlockSpec((R, R), lambda i: (0, 0))],
        out_specs=pl.BlockSpec((XBK, R), lambda i: (i, 0)),
        out_shape=jax.ShapeDtypeStruct((V, R), jnp.float32),
        compiler_params=pltpu.CompilerParams(
            fuse_transposed_lhs_in_matmul=True),
    )


def _sc_gather_body(k_hbm, emb0_hbm, emb1_hbm, e0_hbm, e1_hbm,
                    idx_vmem, buf0, buf1, sem0, sem1):
    wid = lax.axis_index("s") * NC + lax.axis_index("c")
    base = wid * BPW
    pltpu.sync_copy(k_hbm.at[pl.ds(base, BPW)], idx_vmem.at[pl.ds(0, BPW)])
    pltpu.sync_copy(k_hbm.at[pl.ds(B + base, BPW)],
                    idx_vmem.at[pl.ds(BPW, BPW)])

    def fire(voff, emb, buf, sem):
        def gath(g, carry):
            v = idx_vmem[pl.ds(voff + g * 16, 16)]
            for j in range(16):
                pltpu.async_copy(emb.at[pl.ds(v[j], 1)],
                                 buf.at[pl.ds(g * 16 + j, 1)], sem)
            return carry
        lax.fori_loop(0, HB // 16, gath, 0)

    def drain(emb, buf, sem):
        pltpu.make_async_copy(emb.at[pl.ds(0, HB)], buf, sem).wait()

    units = [(0, emb0_hbm, e0_hbm, 0), (HB, emb0_hbm, e0_hbm, HB),
             (BPW, emb1_hbm, e1_hbm, 0), (BPW + HB, emb1_hbm, e1_hbm, HB)]
    bufs = (buf0, buf1)
    sems = (sem0, sem1)
    for u, (voff, emb, _, _) in enumerate(units):
        s = u % 2
        if u >= 2:
            pemb, pout, poff = units[u - 2][1], units[u - 2][2], units[u - 2][3]
            drain(pemb, bufs[s], sems[s])
            pltpu.sync_copy(bufs[s], pout.at[pl.ds(base + poff, HB)])
        fire(voff, emb, bufs[s], sems[s])
    for u in (2, 3):
        s = u % 2
        emb, out, off = units[u][1], units[u][2], units[u][3]
        drain(emb, bufs[s], sems[s])
        pltpu.sync_copy(bufs[s], out.at[pl.ds(base + off, HB)])


@functools.lru_cache(maxsize=None)
def _make_sc_gather():
    return pl.kernel(
        _sc_gather_body,
        out_type=(jax.ShapeDtypeStruct((B, R), jnp.float32),
                  jax.ShapeDtypeStruct((B, R), jnp.float32)),
        mesh=plsc.VectorSubcoreMesh(core_axis_name="c", subcore_axis_name="s",
                                    num_cores=NC, num_subcores=NS),
        scratch_types=[
            pltpu.VMEM((2 * BPW,), jnp.int32),
            pltpu.VMEM((HB, R), jnp.float32),
            pltpu.VMEM((HB, R), jnp.float32),
            pltpu.SemaphoreType.DMA,
            pltpu.SemaphoreType.DMA,
        ],
        compiler_params=pltpu.CompilerParams(use_tc_tiling_on_sc=True),
    )


def _dense_body(ll_ref, e0_ref, e1_ref, w0_ref, a0_ref, a1_ref, b1_ref,
                wmu_ref, bmu_ref, wlv_ref, blv_ref, mu_ref, lv_ref):
    e0 = e0_ref[...]
    e1 = e1_ref[...]
    h = (ll_ref[...] * w0_ref[...]
         + jnp.dot(e0, a0_ref[...], preferred_element_type=jnp.float32)
         + jnp.dot(e1, a1_ref[...], preferred_element_type=jnp.float32)
         + b1_ref[...])
    h = jnp.where(h >= 0, h, ALPHA * h)
    mu_ref[...] = (jnp.dot(h, wmu_ref[...], preferred_element_type=jnp.float32)
                   + bmu_ref[...] + e0 + e1)
    lv_ref[...] = (jnp.dot(h, wlv_ref[...], preferred_element_type=jnp.float32)
                   + blv_ref[...])


def _dense(ll, e0, e1, w0, a0, a1, b1, wmu, bmu, wlv, blv, blk=2048):
    grid = B // blk
    row_spec = lambda w: pl.BlockSpec((blk, w), lambda i: (i, 0))
    full = lambda s: pl.BlockSpec(s, lambda i: (0, 0))
    return pl.pallas_call(
        _dense_body,
        grid=(grid,),
        in_specs=[
            row_spec(1), row_spec(R), row_spec(R),
            full((1, RP)), full((R, RP)), full((R, RP)), full((1, RP)),
            full((RP, R)), full((1, R)), full((RP, R)), full((1, R)),
        ],
        out_specs=[row_spec(R), row_spec(R)],
        out_shape=[jax.ShapeDtypeStruct((B, R), jnp.float32),
                   jax.ShapeDtypeStruct((B, R), jnp.float32)],
    )(ll, e0, e1, w0, a0, a1, b1, wmu, bmu, wlv, blv)


def kernel(log_lib, K, emb0, emb1, W1, b1, Wmu, bmu, Wlv, blv):
    xpose = _make_xpose()
    eye = jnp.eye(R, dtype=jnp.float32)
    emb0r = xpose(emb0.T, eye)
    emb1r = xpose(emb1.T, eye)
    e0, e1 = _make_sc_gather()(K.reshape(2 * B), emb0r, emb1r)
    w0 = W1[:, 0:1].T
    a0 = W1[:, 1:1 + R].T
    a1 = W1[:, 1 + R:1 + 2 * R].T
    mu, lv = _dense(log_lib.reshape(B, 1), e0, e1, w0, a0, a1,
                    b1.reshape(1, RP), Wmu.T, bmu.reshape(1, R),
                    Wlv.T, blv.reshape(1, R))
    return mu, lv


# speedup vs baseline: 1.9269x; 1.0078x over previous
import functools

import jax
import jax.numpy as jnp
from jax import lax
from jax.experimental import pallas as pl
from jax.experimental.pallas import tpu as pltpu
from jax.experimental.pallas import tpu_sc as plsc

B = 16384
V = 1000000
R = 64
RP = 128
ALPHA = 0.01

NC = 2
NS = 16
NW = NC * NS
BPW = B // NW
HB = BPW // 2

XBK = 32000


def _xpose_body(in_ref, eye_ref, out_ref):
    out_ref[...] = lax.dot_general(
        in_ref[...], eye_ref[...], (((0,), (0,)), ((), ())),
        preferred_element_type=jnp.float32)


@functools.lru_cache(maxsize=None)
def _make_xpose():
    grid = (V + XBK - 1) // XBK
    return pl.pallas_call(
        _xpose_body,
        grid=(grid,),
        in_specs=[pl.BlockSpec((R, XBK), lambda i: (0, i)),
                  pl.BlockSpec((R, R), lambda i: (0, 0))],
        out_specs=pl.BlockSpec((XBK, R), lambda i: (i, 0)),
        out_shape=jax.ShapeDtypeStruct((V, R), jnp.float32),
        compiler_params=pltpu.CompilerParams(
            fuse_transposed_lhs_in_matmul=True),
    )


def _sc_gather_body(k_hbm, emb_hbm, e_hbm, idx_vmem, buf0, buf1, sem0, sem1):
    wid = lax.axis_index("s") * NC + lax.axis_index("c")
    base = wid * BPW
    pltpu.sync_copy(k_hbm.at[pl.ds(base, BPW)], idx_vmem)

    def fire(voff, buf, sem):
        def gath(g, carry):
            v = idx_vmem[pl.ds(voff + g * 16, 16)]
            for j in range(16):
                pltpu.async_copy(emb_hbm.at[pl.ds(v[j], 1)],
                                 buf.at[pl.ds(g * 16 + j, 1)], sem)
            return carry
        lax.fori_loop(0, HB // 16, gath, 0)

    def drain(buf, sem):
        pltpu.make_async_copy(emb_hbm.at[pl.ds(0, HB)], buf, sem).wait()

    fire(0, buf0, sem0)
    fire(HB, buf1, sem1)
    drain(buf0, sem0)
    pltpu.sync_copy(buf0, e_hbm.at[pl.ds(base, HB)])
    drain(buf1, sem1)
    pltpu.sync_copy(buf1, e_hbm.at[pl.ds(base + HB, HB)])


@functools.lru_cache(maxsize=None)
def _make_sc_gather():
    return pl.kernel(
        _sc_gather_body,
        out_type=jax.ShapeDtypeStruct((B, R), jnp.float32),
        mesh=plsc.VectorSubcoreMesh(core_axis_name="c", subcore_axis_name="s",
                                    num_cores=NC, num_subcores=NS),
        scratch_types=[
            pltpu.VMEM((BPW,), jnp.int32),
            pltpu.VMEM((HB, R), jnp.float32),
            pltpu.VMEM((HB, R), jnp.float32),
            pltpu.SemaphoreType.DMA,
            pltpu.SemaphoreType.DMA,
        ],
        compiler_params=pltpu.CompilerParams(use_tc_tiling_on_sc=True),
    )


def _dense_body(ll_ref, e0_ref, e1_ref, w0_ref, a0_ref, a1_ref, b1_ref,
                wmu_ref, bmu_ref, wlv_ref, blv_ref, mu_ref, lv_ref):
    e0 = e0_ref[...]
    e1 = e1_ref[...]
    h = (ll_ref[...] * w0_ref[...]
         + jnp.dot(e0, a0_ref[...], preferred_element_type=jnp.float32)
         + jnp.dot(e1, a1_ref[...], preferred_element_type=jnp.float32)
         + b1_ref[...])
    h = jnp.where(h >= 0, h, ALPHA * h)
    mu_ref[...] = (jnp.dot(h, wmu_ref[...], preferred_element_type=jnp.float32)
                   + bmu_ref[...] + e0 + e1)
    lv_ref[...] = (jnp.dot(h, wlv_ref[...], preferred_element_type=jnp.float32)
                   + blv_ref[...])


def _dense(ll, e0, e1, w0, a0, a1, b1, wmu, bmu, wlv, blv, blk=2048):
    grid = B // blk
    row_spec = lambda w: pl.BlockSpec((blk, w), lambda i: (i, 0))
    full = lambda s: pl.BlockSpec(s, lambda i: (0, 0))
    return pl.pallas_call(
        _dense_body,
        grid=(grid,),
        in_specs=[
            row_spec(1), row_spec(R), row_spec(R),
            full((1, RP)), full((R, RP)), full((R, RP)), full((1, RP)),
            full((RP, R)), full((1, R)), full((RP, R)), full((1, R)),
        ],
        out_specs=[row_spec(R), row_spec(R)],
        out_shape=[jax.ShapeDtypeStruct((B, R), jnp.float32),
                   jax.ShapeDtypeStruct((B, R), jnp.float32)],
    )(ll, e0, e1, w0, a0, a1, b1, wmu, bmu, wlv, blv)


def kernel(log_lib, K, emb0, emb1, W1, b1, Wmu, bmu, Wlv, blv):
    xpose = _make_xpose()
    gather = _make_sc_gather()
    eye = jnp.eye(R, dtype=jnp.float32)
    emb0r = xpose(emb0.T, eye)
    e0 = gather(K[0], emb0r)
    emb1r = xpose(emb1.T, eye)
    e1 = gather(K[1], emb1r)
    w0 = W1[:, 0:1].T
    a0 = W1[:, 1:1 + R].T
    a1 = W1[:, 1 + R:1 + 2 * R].T
    mu, lv = _dense(log_lib.reshape(B, 1), e0, e1, w0, a0, a1,
                    b1.reshape(1, RP), Wmu.T, bmu.reshape(1, R),
                    Wlv.T, blv.reshape(1, R))
    return mu, lv


# speedup vs baseline: 1.9298x; 1.0015x over previous
import functools

import jax
import jax.numpy as jnp
from jax import lax
from jax.experimental import pallas as pl
from jax.experimental.pallas import tpu as pltpu
from jax.experimental.pallas import tpu_sc as plsc

B = 16384
V = 1000000
R = 64
RP = 128
ALPHA = 0.01

NC = 2
NS = 16
NW = NC * NS
BPW = B // NW
HB = BPW // 2

XBK = 32000


def _xpose_body(in_ref, eye_ref, out_ref):
    x = in_ref[...]
    hi = x.astype(jnp.bfloat16)
    lo = (x - hi.astype(jnp.float32)).astype(jnp.bfloat16)
    eye = eye_ref[...].astype(jnp.bfloat16)
    dims = (((0,), (0,)), ((), ()))
    out_ref[...] = (
        lax.dot_general(hi, eye, dims, preferred_element_type=jnp.float32)
        + lax.dot_general(lo, eye, dims, preferred_element_type=jnp.float32))


@functools.lru_cache(maxsize=None)
def _make_xpose():
    grid = (V + XBK - 1) // XBK
    return pl.pallas_call(
        _xpose_body,
        grid=(grid,),
        in_specs=[pl.BlockSpec((R, XBK), lambda i: (0, i)),
                  pl.BlockSpec((R, R), lambda i: (0, 0))],

        out_specs=pl.BlockSpec((XBK, R), lambda i: (i, 0)),
        out_shape=jax.ShapeDtypeStruct((V, R), jnp.float32),
        compiler_params=pltpu.CompilerParams(
            fuse_transposed_lhs_in_matmul=True),
    )


def _sc_gather_body(k_hbm, emb_hbm, e_hbm, idx_vmem, buf0, buf1, sem0, sem1):
    wid = lax.axis_index("s") * NC + lax.axis_index("c")
    base = wid * BPW
    pltpu.sync_copy(k_hbm.at[pl.ds(base, BPW)], idx_vmem)

    def fire(voff, buf, sem):
        def gath(g, carry):
            v = idx_vmem[pl.ds(voff + g * 16, 16)]
            for j in range(16):
                pltpu.async_copy(emb_hbm.at[pl.ds(v[j], 1)],
                                 buf.at[pl.ds(g * 16 + j, 1)], sem)
            return carry
        lax.fori_loop(0, HB // 16, gath, 0)

    def drain(buf, sem):
        pltpu.make_async_copy(emb_hbm.at[pl.ds(0, HB)], buf, sem).wait()

    fire(0, buf0, sem0)
    fire(HB, buf1, sem1)
    drain(buf0, sem0)
    pltpu.sync_copy(buf0, e_hbm.at[pl.ds(base, HB)])
    drain(buf1, sem1)
    pltpu.sync_copy(buf1, e_hbm.at[pl.ds(base + HB, HB)])


@functools.lru_cache(maxsize=None)
def _make_sc_gather():
    return pl.kernel(
        _sc_gather_body,
        out_type=jax.ShapeDtypeStruct((B, R), jnp.float32),
        mesh=plsc.VectorSubcoreMesh(core_axis_name="c", subcore_axis_name="s",
                                    num_cores=NC, num_subcores=NS),
        scratch_types=[
            pltpu.VMEM((BPW,), jnp.int32),
            pltpu.VMEM((HB, R), jnp.float32),
            pltpu.VMEM((HB, R), jnp.float32),
            pltpu.SemaphoreType.DMA,
            pltpu.SemaphoreType.DMA,
        ],
        compiler_params=pltpu.CompilerParams(use_tc_tiling_on_sc=True),
    )


def _dense_body(ll_ref, e0_ref, e1_ref, w0_ref, a0_ref, a1_ref, b1_ref,
                wmu_ref, bmu_ref, wlv_ref, blv_ref, mu_ref, lv_ref):
    e0 = e0_ref[...]
    e1 = e1_ref[...]
    h = (ll_ref[...] * w0_ref[...]
         + jnp.dot(e0, a0_ref[...], preferred_element_type=jnp.float32)
         + jnp.dot(e1, a1_ref[...], preferred_element_type=jnp.float32)
         + b1_ref[...])
    h = jnp.where(h >= 0, h, ALPHA * h)
    mu_ref[...] = (jnp.dot(h, wmu_ref[...], preferred_element_type=jnp.float32)
                   + bmu_ref[...] + e0 + e1)
    lv_ref[...] = (jnp.dot(h, wlv_ref[...], preferred_element_type=jnp.float32)
                   + blv_ref[...])


def _dense(ll, e0, e1, w0, a0, a1, b1, wmu, bmu, wlv, blv, blk=2048):
    grid = B // blk
    row_spec = lambda w: pl.BlockSpec((blk, w), lambda i: (i, 0))
    full = lambda s: pl.BlockSpec(s, lambda i: (0, 0))
    return pl.pallas_call(
        _dense_body,
        grid=(grid,),
        in_specs=[
            row_spec(1), row_spec(R), row_spec(R),
            full((1, RP)), full((R, RP)), full((R, RP)), full((1, RP)),
            full((RP, R)), full((1, R)), full((RP, R)), full((1, R)),
        ],
        out_specs=[row_spec(R), row_spec(R)],
        out_shape=[jax.ShapeDtypeStruct((B, R), jnp.float32),
                   jax.ShapeDtypeStruct((B, R), jnp.float32)],
    )(ll, e0, e1, w0, a0, a1, b1, wmu, bmu, wlv, blv)


def kernel(log_lib, K, emb0, emb1, W1, b1, Wmu, bmu, Wlv, blv):
    xpose = _make_xpose()
    gather = _make_sc_gather()
    eye = jnp.eye(R, dtype=jnp.float32)
    emb0r = xpose(emb0.T, eye)
    e0 = gather(K[0], emb0r)
    emb1r = xpose(emb1.T, eye)
    e1 = gather(K[1], emb1r)
    w0 = W1[:, 0:1].T
    a0 = W1[:, 1:1 + R].T
    a1 = W1[:, 1 + R:1 + 2 * R].T
    mu, lv = _dense(log_lib.reshape(B, 1), e0, e1, w0, a0, a1,
                    b1.reshape(1, RP), Wmu.T, bmu.reshape(1, R),
                    Wlv.T, blv.reshape(1, R))
    return mu, lv
